# baseline (speedup 1.0000x reference)
import jax
import jax.numpy as jnp
from jax.experimental import pallas as pl
from jax.experimental.pallas import tpu as pltpu

_ROW_TILE = 3200


def _fused_heads_kernel(x_ref, w_ref, b_ref, o_ref):
    o_ref[...] = (
        jnp.dot(x_ref[...], w_ref[...], preferred_element_type=jnp.float32)
        + b_ref[...]
    )


def kernel(local_features, W_coords, b_coords, W_pres, b_pres):
    B, C, R, D = local_features.shape
    M = B * C * R
    x = local_features.reshape(M, D)
    w = jnp.concatenate([W_coords, W_pres], axis=0).T
    b = jnp.concatenate([b_coords, b_pres], axis=0).reshape(1, 5)

    tile = _ROW_TILE if M % _ROW_TILE == 0 else M
    grid = (M // tile,)

    out = pl.pallas_call(
        _fused_heads_kernel,
        grid=grid,
        in_specs=[
            pl.BlockSpec((tile, D), lambda i: (i, 0)),
            pl.BlockSpec((D, 5), lambda i: (0, 0)),
            pl.BlockSpec((1, 5), lambda i: (0, 0)),
        ],
        out_specs=pl.BlockSpec((tile, 5), lambda i: (i, 0)),
        out_shape=jax.ShapeDtypeStruct((M, 5), jnp.float32),
        compiler_params=pltpu.CompilerParams(
            dimension_semantics=("parallel",),
        ),
    )(x, w, b)

    out = out.reshape(B, C, R, 5)
    return (out[..., :4], out[..., 4:])


# speedup vs baseline: 1.0090x; 1.0090x over previous
import jax
import jax.numpy as jnp
from jax.experimental import pallas as pl
from jax.experimental.pallas import tpu as pltpu

_STREAMS = 4
_ROW_TILE = 1600


def _fused_heads_kernel(x0_ref, x1_ref, x2_ref, x3_ref, w_ref, b_ref, o_ref):
    w = w_ref[...]
    b = b_ref[...]
    t = _ROW_TILE
    for j, x_ref in enumerate((x0_ref, x1_ref, x2_ref, x3_ref)):
        o_ref[pl.ds(j * t, t), :] = (
            jnp.dot(x_ref[...], w, preferred_element_type=jnp.float32) + b
        )


def kernel(local_features, W_coords, b_coords, W_pres, b_pres):
    B, C, R, D = local_features.shape
    M = B * C * R
    x = local_features.reshape(M, D)
    w = jnp.concatenate([W_coords, W_pres], axis=0).T
    b = jnp.concatenate([b_coords, b_pres], axis=0).reshape(1, 5)

    S, tile = _STREAMS, _ROW_TILE
    grid = (M // (S * tile),)

    def x_map(j):
        return lambda i: (S * i + j, 0)

    out = pl.pallas_call(
        _fused_heads_kernel,
        grid=grid,
        in_specs=[pl.BlockSpec((tile, D), x_map(j)) for j in range(S)]
        + [
            pl.BlockSpec((D, 5), lambda i: (0, 0)),
            pl.BlockSpec((1, 5), lambda i: (0, 0)),
        ],
        out_specs=pl.BlockSpec((S * tile, 5), lambda i: (i, 0)),
        out_shape=jax.ShapeDtypeStruct((M, 5), jnp.float32),
        compiler_params=pltpu.CompilerParams(
            dimension_semantics=("arbitrary",),
        ),
    )(x, x, x, x, w, b)

    out = out.reshape(B, C, R, 5)
    return (out[..., :4], out[..., 4:])
